# Initial kernel scaffold; baseline (speedup 1.0000x reference)
import jax
import jax.numpy as jnp
from jax.experimental import pallas as pl


def kernel(inp, history, bbox_emb, nodes_min, nodes_extent, bbox_dims):
    raise NotImplementedError("write your pallas kernel here")


# speedup vs baseline: 4.4389x; 4.4389x over previous
import functools

import jax
import jax.numpy as jnp
from jax import lax
from jax.experimental import pallas as pl
from jax.experimental.pallas import tpu as pltpu
from jax.experimental.pallas import tpu_sc as plsc

_PI1 = 774363409
_PI2 = -1640531535
_PI3 = 100000007

_NC = 2
_NS = 16
_NW = _NC * _NS

_SEG_PER_PHASE = 16
_PHASES = 2
_G = _SEG_PER_PHASE * _PHASES // 4


@functools.partial(jax.jit, static_argnums=(4,))
def _sc_encode(inp_t, hist_flat, bbox_emb, params, interpret=False):
    R, _, P = inp_t.shape
    T, D = bbox_emb.shape
    mask = T - 1
    rpw = R // _NW
    chunks = rpw // _G
    pi1 = jnp.int32(_PI1)
    pi2 = jnp.int32(_PI2)
    pi3 = jnp.int32(_PI3)
    mesh = plsc.VectorSubcoreMesh(core_axis_name="c", subcore_axis_name="s",
                                  num_cores=_NC, num_subcores=_NS)

    @functools.partial(
        pl.kernel,
        out_type=jax.ShapeDtypeStruct((R, 4, P, D), jnp.float32),
        mesh=mesh,
        scratch_types=[
            pltpu.VMEM((_G, 3, P), jnp.float32),
            pltpu.VMEM((_G * 4,), jnp.int32),
            pltpu.VMEM((_G * 4, 16), jnp.float32),
            pltpu.VMEM((_PHASES, _SEG_PER_PHASE, 8 * P), jnp.int32),
            pltpu.VMEM((_PHASES, _SEG_PER_PHASE, 8 * P), jnp.float32),
            pltpu.VMEM((_PHASES, _SEG_PER_PHASE, 8 * P, D), jnp.float32),
            pltpu.VMEM((_G, 4, P, D), jnp.float32),
            pltpu.SemaphoreType.DMA,
            pltpu.SemaphoreType.DMA,
        ],
        compiler_params=pltpu.CompilerParams(use_tc_tiling_on_sc=False),
        interpret=interpret,
    )
    def k(inp_hbm, hist_hbm, emb_hbm, par_hbm, out_hbm,
          inp_v, hist_v, params_v, idx_v, w_v, gath_v, out_v, sem0, sem1):
        wid = lax.axis_index("s") * _NC + lax.axis_index("c")
        sems = [sem0, sem1]

        def gen_phase(ph):
            def seg_body(s, carry):
                g = ph * _SEG_PER_PHASE + s
                rl = lax.shift_right_logical(g, 2)
                pr = params_v[g, :]
                mn0 = pr[0]
                mn1 = pr[1]
                mn2 = pr[2]
                ex0 = pr[3]
                ex1 = pr[4]
                ex2 = pr[5]
                bd0 = pr[6]
                bd1 = pr[7]
                bd2 = pr[8]
                b = pr[9].astype(jnp.int32)
                px = inp_v[rl, 0, :]
                py = inp_v[rl, 1, :]
                pz = inp_v[rl, 2, :]

                def axis(pp, mn, ex, bd):
                    v = (pp - mn) / ex
                    v = jnp.clip(v, 1e-6, 1.0 - 1e-6) * bd
                    i0 = v.astype(jnp.int32)
                    f0 = i0.astype(jnp.float32)
                    fr = v - f0
                    i1 = jnp.where(v > f0, i0 + 1, i0)
                    return i0, i1, fr

                x0, x1, fx = axis(px, mn0, ex0, bd0)
                y0, y1, fy = axis(py, mn1, ex1, bd1)
                z0, z1, fz = axis(pz, mn2, ex2, bd2)
                hy0 = y0 * pi1
                hy1 = y1 * pi1
                zb0 = (z0 * pi2) ^ (b * pi3)
                zb1 = (z1 * pi2) ^ (b * pi3)
                t00 = x0 ^ hy0
                t01 = x0 ^ hy1
                t10 = x1 ^ hy0
                t11 = x1 ^ hy1
                wx0 = 1.0 - fx
                wy0 = 1.0 - fy
                wz0 = 1.0 - fz
                wxy00 = wx0 * wy0
                wxy01 = wx0 * fy
                wxy10 = fx * wy0
                wxy11 = fx * fy
                idxs = (t00 ^ zb0, t00 ^ zb1, t01 ^ zb0, t01 ^ zb1,
                        t10 ^ zb0, t10 ^ zb1, t11 ^ zb0, t11 ^ zb1)
                ws = (wxy00 * wz0, wxy00 * fz, wxy01 * wz0, wxy01 * fz,
                      wxy10 * wz0, wxy10 * fz, wxy11 * wz0, wxy11 * fz)
                for c in range(8):
                    idx_v[ph, s, pl.ds(c * P, P)] = idxs[c] & mask
                    w_v[ph, s, pl.ds(c * P, P)] = ws[c]
                pltpu.async_copy(emb_hbm.at[idx_v.at[ph, s]],
                                 gath_v.at[ph, s], sems[ph])
                return carry

            lax.fori_loop(0, _SEG_PER_PHASE, seg_body, 0)

        def combine_phase(ph):
            for s in range(_SEG_PER_PHASE):
                pltpu.make_async_copy(emb_hbm.at[idx_v.at[ph, s]],
                                      gath_v.at[ph, s], sems[ph]).wait()

            def seg_body(s, carry):
                g = ph * _SEG_PER_PHASE + s
                rl = lax.shift_right_logical(g, 2)
                dp = lax.bitwise_and(g, 3)
                wrows = [w_v[ph, s, pl.ds(c * P, P)] for c in range(8)]
                for p in range(P):
                    acc = wrows[0][p] * gath_v[ph, s, p, :]
                    for c in range(1, 8):
                        acc = acc + wrows[c][p] * gath_v[ph, s, c * P + p, :]
                    out_v[rl, dp, p, :] = acc
                return carry

            lax.fori_loop(0, _SEG_PER_PHASE, seg_body, 0)

        def chunk_body(ci, carry):
            ray0 = wid * rpw + ci * _G
            pltpu.sync_copy(inp_hbm.at[pl.ds(ray0, _G)], inp_v)
            pltpu.sync_copy(hist_hbm.at[pl.ds(ray0 * 4, _G * 4)], hist_v)
            pltpu.async_copy(par_hbm.at[hist_v], params_v, sem0).wait()
            gen_phase(0)
            gen_phase(1)
            combine_phase(0)
            combine_phase(1)
            pltpu.sync_copy(out_v, out_hbm.at[pl.ds(ray0, _G)])
            return carry

        lax.fori_loop(0, chunks, chunk_body, 0)

    return k(inp_t, hist_flat, bbox_emb, params)


def kernel(inp, history, bbox_emb, nodes_min, nodes_extent, bbox_dims):
    n_rays, n_points, _ = inp.shape
    n_nodes = nodes_min.shape[0]
    inp_t = jnp.transpose(inp, (0, 2, 1))
    hist_flat = history.reshape(-1).astype(jnp.int32)
    params = jnp.concatenate(
        [nodes_min, nodes_extent, bbox_dims,
         jnp.arange(n_nodes, dtype=jnp.float32)[:, None],
         jnp.zeros((n_nodes, 6), jnp.float32)], axis=1)
    out = _sc_encode(inp_t, hist_flat, bbox_emb, params)
    return out.reshape(n_rays, 4 * n_points * bbox_emb.shape[1])


# speedup vs baseline: 5.7826x; 1.3027x over previous
import functools

import jax
import jax.numpy as jnp
from jax import lax
from jax.experimental import pallas as pl
from jax.experimental.pallas import tpu as pltpu
from jax.experimental.pallas import tpu_sc as plsc

_PI1 = 774363409
_PI2 = -1640531535
_PI3 = 100000007

_NC = 2
_NS = 16
_NW = _NC * _NS

_SEG_PER_PHASE = 16
_PHASES = 2
_G = _SEG_PER_PHASE * _PHASES // 4
_SEG_PER_CHUNK = 4 * _G


@functools.partial(jax.jit, static_argnums=(4,))
def _sc_encode(inp_t, hist_flat, bbox_emb, params, interpret=False):
    R, _, P = inp_t.shape
    T, D = bbox_emb.shape
    mask = T - 1
    rpw = R // _NW
    chunks = rpw // _G
    pi1 = jnp.int32(_PI1)
    pi2 = jnp.int32(_PI2)
    pi3 = jnp.int32(_PI3)
    mesh = plsc.VectorSubcoreMesh(core_axis_name="c", subcore_axis_name="s",
                                  num_cores=_NC, num_subcores=_NS)

    @functools.partial(
        pl.kernel,
        out_type=jax.ShapeDtypeStruct((R * 4 * P * D,), jnp.float32),
        mesh=mesh,
        scratch_types=[
            pltpu.VMEM((rpw, 3, P), jnp.float32),
            pltpu.VMEM((rpw * 4,), jnp.int32),
            pltpu.VMEM((2, _SEG_PER_CHUNK, 16), jnp.float32),
            pltpu.VMEM((_PHASES, _SEG_PER_PHASE, 8 * P), jnp.int32),
            pltpu.VMEM((_PHASES, _SEG_PER_PHASE, 8 * P), jnp.float32),
            pltpu.VMEM((_PHASES * _SEG_PER_PHASE * 8 * P, D), jnp.float32),
            pltpu.VMEM((_SEG_PER_CHUNK * P * D,), jnp.float32),
            pltpu.SemaphoreType.DMA,
            pltpu.SemaphoreType.DMA,
            pltpu.SemaphoreType.DMA,
        ],
        compiler_params=pltpu.CompilerParams(use_tc_tiling_on_sc=False,
                                             needs_layout_passes=False),
        interpret=interpret,
    )
    def k(inp_hbm, hist_hbm, emb_hbm, par_hbm, out_hbm,
          inp_all, hist_all, params_v, idx_v, w_v, gath_v, out_v,
          sem0, sem1, psem):
        wid = lax.axis_index("s") * _NC + lax.axis_index("c")
        ray0_t = wid * rpw
        sems = [sem0, sem1]
        iot = lax.iota(jnp.int32, 16)

        pltpu.sync_copy(inp_hbm.at[pl.ds(ray0_t, rpw)], inp_all)
        pltpu.sync_copy(hist_hbm.at[pl.ds(ray0_t * 4, rpw * 4)], hist_all)
        pltpu.async_copy(par_hbm.at[hist_all.at[pl.ds(0, _SEG_PER_CHUNK)]],
                         params_v.at[0], psem)

        def gen_phase(ph, ci, pb):
            def seg_body(s, carry):
                g = ph * _SEG_PER_PHASE + s
                rl = ci * _G + lax.shift_right_logical(g, 2)
                pr = params_v[pb, g, :]
                mn0 = pr[0]
                mn1 = pr[1]
                mn2 = pr[2]
                ex0 = pr[3]
                ex1 = pr[4]
                ex2 = pr[5]
                bd0 = pr[6]
                bd1 = pr[7]
                bd2 = pr[8]
                b = pr[9].astype(jnp.int32)
                px = inp_all[rl, 0, :]
                py = inp_all[rl, 1, :]
                pz = inp_all[rl, 2, :]

                def axis(pp, mn, ex, bd):
                    v = (pp - mn) / ex
                    v = jnp.clip(v, 1e-6, 1.0 - 1e-6) * bd
                    i0 = v.astype(jnp.int32)
                    f0 = i0.astype(jnp.float32)
                    fr = v - f0
                    i1 = jnp.where(v > f0, i0 + 1, i0)
                    return i0, i1, fr

                x0, x1, fx = axis(px, mn0, ex0, bd0)
                y0, y1, fy = axis(py, mn1, ex1, bd1)
                z0, z1, fz = axis(pz, mn2, ex2, bd2)
                hy0 = y0 * pi1
                hy1 = y1 * pi1
                zb0 = (z0 * pi2) ^ (b * pi3)
                zb1 = (z1 * pi2) ^ (b * pi3)
                t00 = x0 ^ hy0
                t01 = x0 ^ hy1
                t10 = x1 ^ hy0
                t11 = x1 ^ hy1
                wx0 = 1.0 - fx
                wy0 = 1.0 - fy
                wz0 = 1.0 - fz
                wxy00 = wx0 * wy0
                wxy01 = wx0 * fy
                wxy10 = fx * wy0
                wxy11 = fx * fy
                idxs = (t00 ^ zb0, t00 ^ zb1, t01 ^ zb0, t01 ^ zb1,
                        t10 ^ zb0, t10 ^ zb1, t11 ^ zb0, t11 ^ zb1)
                ws = (wxy00 * wz0, wxy00 * fz, wxy01 * wz0, wxy01 * fz,
                      wxy10 * wz0, wxy10 * fz, wxy11 * wz0, wxy11 * fz)
                for c in range(8):
                    idx_v[ph, s, pl.ds(c * P, P)] = idxs[c] & mask
                    w_v[ph, s, pl.ds(c * P, P)] = ws[c]
                pltpu.async_copy(
                    emb_hbm.at[idx_v.at[ph, s]],
                    gath_v.at[pl.ds((ph * _SEG_PER_PHASE + s) * (8 * P), 8 * P)],
                    sems[ph])
                return carry

            lax.fori_loop(0, _SEG_PER_PHASE, seg_body, 0)

        def combine_phase(ph):
            for s in range(_SEG_PER_PHASE):
                pltpu.make_async_copy(
                    emb_hbm.at[idx_v.at[ph, s]],
                    gath_v.at[pl.ds((ph * _SEG_PER_PHASE + s) * (8 * P), 8 * P)],
                    sems[ph]).wait()

            def seg_body(s, carry):
                g = ph * _SEG_PER_PHASE + s
                wrows = [w_v[ph, s, pl.ds(c * P, P)] for c in range(8)]
                rowp = (ph * _SEG_PER_PHASE + s) * (8 * P) + iot
                acc = [None] * D
                for c in range(8):
                    rowc = rowp + c * P
                    for d in range(D):
                        gv = plsc.load_gather(
                            gath_v, [rowc, jnp.full((16,), d, jnp.int32)])
                        if c == 0:
                            acc[d] = wrows[0] * gv
                        else:
                            acc[d] = acc[d] + wrows[c] * gv
                ob = g * (P * D) + iot * D
                for d in range(D):
                    plsc.store_scatter(out_v, [ob + d], acc[d])
                return carry

            lax.fori_loop(0, _SEG_PER_PHASE, seg_body, 0)

        def chunk_body(ci, carry):
            pb = lax.bitwise_and(ci, 1)
            pltpu.make_async_copy(
                par_hbm.at[hist_all.at[pl.ds(0, _SEG_PER_CHUNK)]],
                params_v.at[pb], psem).wait()
            gen_phase(0, ci, pb)
            gen_phase(1, ci, pb)

            @pl.when(ci < chunks - 1)
            def _():
                pltpu.async_copy(
                    par_hbm.at[hist_all.at[pl.ds((ci + 1) * _SEG_PER_CHUNK,
                                                 _SEG_PER_CHUNK)]],
                    params_v.at[lax.bitwise_and(ci + 1, 1)], psem)

            combine_phase(0)
            combine_phase(1)
            pltpu.sync_copy(
                out_v,
                out_hbm.at[pl.ds((ray0_t + ci * _G) * (4 * P * D),
                                 _SEG_PER_CHUNK * P * D)])
            return carry

        lax.fori_loop(0, chunks, chunk_body, 0)

    return k(inp_t, hist_flat, bbox_emb, params)


def kernel(inp, history, bbox_emb, nodes_min, nodes_extent, bbox_dims):
    n_rays, n_points, _ = inp.shape
    n_nodes = nodes_min.shape[0]
    inp_t = jnp.transpose(inp, (0, 2, 1))
    hist_flat = history.reshape(-1).astype(jnp.int32)
    params = jnp.concatenate(
        [nodes_min, nodes_extent, bbox_dims,
         jnp.arange(n_nodes, dtype=jnp.float32)[:, None],
         jnp.zeros((n_nodes, 6), jnp.float32)], axis=1)
    out = _sc_encode(inp_t, hist_flat, bbox_emb, params)
    return out.reshape(n_rays, 4 * n_points * bbox_emb.shape[1])


# speedup vs baseline: 6.5139x; 1.1265x over previous
import functools

import jax
import jax.numpy as jnp
from jax import lax
from jax.experimental import pallas as pl
from jax.experimental.pallas import tpu as pltpu
from jax.experimental.pallas import tpu_sc as plsc

_PI1 = 774363409
_PI2 = -1640531535
_PI3 = 100000007

_NC = 2
_NS = 16
_NW = _NC * _NS

_SPP = 8
_NB = 4


@functools.partial(jax.jit, static_argnums=(4,))
def _sc_encode(inp_t, hist_flat, bbox_emb, params, interpret=False):
    R, _, P = inp_t.shape
    T, D = bbox_emb.shape
    mask = T - 1
    rpw = R // _NW
    F = rpw * 4 // _SPP
    pi1 = jnp.int32(_PI1)
    pi2 = jnp.int32(_PI2)
    pi3 = jnp.int32(_PI3)
    mesh = plsc.VectorSubcoreMesh(core_axis_name="c", subcore_axis_name="s",
                                  num_cores=_NC, num_subcores=_NS)

    @functools.partial(
        pl.kernel,
        out_type=jax.ShapeDtypeStruct((R * 4 * P * D,), jnp.float32),
        mesh=mesh,
        scratch_types=[
            pltpu.VMEM((rpw, 3, P), jnp.float32),
            pltpu.VMEM((rpw * 4,), jnp.int32),
            pltpu.VMEM((2, _SPP, 16), jnp.float32),
            pltpu.VMEM((_NB, _SPP, 8 * P), jnp.int32),
            pltpu.VMEM((_NB, _SPP, 8 * P), jnp.float32),
            pltpu.VMEM((_NB * _SPP * 8 * P, D), jnp.float32),
            pltpu.VMEM((_SPP * P * D,), jnp.float32),
            pltpu.SemaphoreType.DMA,
            pltpu.SemaphoreType.DMA,
            pltpu.SemaphoreType.DMA,
            pltpu.SemaphoreType.DMA,
            pltpu.SemaphoreType.DMA,
            pltpu.SemaphoreType.DMA,
        ],
        compiler_params=pltpu.CompilerParams(use_tc_tiling_on_sc=False,
                                             needs_layout_passes=False),
        interpret=interpret,
    )
    def k(inp_hbm, hist_hbm, emb_hbm, par_hbm, out_hbm,
          inp_all, hist_all, params_v, idx_v, w_v, gath_v, out_v,
          sem0, sem1, sem2, sem3, psem, osem):
        wid = lax.axis_index("s") * _NC + lax.axis_index("c")
        ray0_t = wid * rpw
        seg0_t = ray0_t * 4
        sems = [sem0, sem1, sem2, sem3]
        iot = lax.iota(jnp.int32, 16)
        iot16 = iot * D

        pltpu.sync_copy(inp_hbm.at[pl.ds(ray0_t, rpw)], inp_all)
        pltpu.sync_copy(hist_hbm.at[pl.ds(seg0_t, rpw * 4)], hist_all)

        def issue_params(p):
            pltpu.async_copy(
                par_hbm.at[hist_all.at[pl.ds(p * _SPP, _SPP)]],
                params_v.at[p % 2 if isinstance(p, int) else lax.bitwise_and(p, 1)],
                psem)

        def wait_params(p):
            pltpu.make_async_copy(
                par_hbm.at[hist_all.at[pl.ds(p * _SPP, _SPP)]],
                params_v.at[p % 2 if isinstance(p, int) else lax.bitwise_and(p, 1)],
                psem).wait()

        def gen(p):
            static = isinstance(p, int)
            rb = p % _NB if static else lax.rem(p, _NB)
            pb = p % 2 if static else lax.bitwise_and(p, 1)

            def seg_body(j, carry):
                s = p * _SPP + j
                rl = lax.shift_right_logical(s, 2)
                pr = params_v[pb, j, :]
                mn0 = pr[0]
                mn1 = pr[1]
                mn2 = pr[2]
                ex0 = pr[3]
                ex1 = pr[4]
                ex2 = pr[5]
                bd0 = pr[6]
                bd1 = pr[7]
                bd2 = pr[8]
                b = pr[9].astype(jnp.int32)
                px = inp_all[rl, 0, :]
                py = inp_all[rl, 1, :]
                pz = inp_all[rl, 2, :]

                def axis(pp, mn, ex, bd):
                    v = (pp - mn) / ex
                    v = jnp.clip(v, 1e-6, 1.0 - 1e-6) * bd
                    i0 = v.astype(jnp.int32)
                    f0 = i0.astype(jnp.float32)
                    fr = v - f0
                    i1 = jnp.where(v > f0, i0 + 1, i0)
                    return i0, i1, fr

                x0, x1, fx = axis(px, mn0, ex0, bd0)
                y0, y1, fy = axis(py, mn1, ex1, bd1)
                z0, z1, fz = axis(pz, mn2, ex2, bd2)
                hy0 = y0 * pi1
                hy1 = y1 * pi1
                zb0 = (z0 * pi2) ^ (b * pi3)
                zb1 = (z1 * pi2) ^ (b * pi3)
                t00 = x0 ^ hy0
                t01 = x0 ^ hy1
                t10 = x1 ^ hy0
                t11 = x1 ^ hy1
                wx0 = 1.0 - fx
                wy0 = 1.0 - fy
                wz0 = 1.0 - fz
                wxy00 = wx0 * wy0
                wxy01 = wx0 * fy
                wxy10 = fx * wy0
                wxy11 = fx * fy
                idxs = (t00 ^ zb0, t00 ^ zb1, t01 ^ zb0, t01 ^ zb1,
                        t10 ^ zb0, t10 ^ zb1, t11 ^ zb0, t11 ^ zb1)
                ws = (wxy00 * wz0, wxy00 * fz, wxy01 * wz0, wxy01 * fz,
                      wxy10 * wz0, wxy10 * fz, wxy11 * wz0, wxy11 * fz)
                for c in range(8):
                    idx_v[rb, j, pl.ds(c * P, P)] = idxs[c] & mask
                    w_v[rb, j, pl.ds(c * P, P)] = ws[c]
                if static:
                    pltpu.async_copy(
                        emb_hbm.at[idx_v.at[rb, j]],
                        gath_v.at[pl.ds((rb * _SPP + j) * (8 * P), 8 * P)],
                        sems[rb])
                else:
                    for bb in range(_NB):
                        @pl.when(rb == bb)
                        def _():
                            pltpu.async_copy(
                                emb_hbm.at[idx_v.at[rb, j]],
                                gath_v.at[pl.ds((rb * _SPP + j) * (8 * P),
                                                8 * P)],
                                sems[bb])
                return carry

            lax.fori_loop(0, _SPP, seg_body, 0)

        def drain_gathers(rb_static, rb):
            for j in range(_SPP):
                pltpu.make_async_copy(
                    emb_hbm.at[idx_v.at[rb, j]],
                    gath_v.at[pl.ds((rb * _SPP + j) * (8 * P), 8 * P)],
                    sems[rb_static]).wait()

        def combine(p):
            static = isinstance(p, int)
            rb = p % _NB if static else lax.rem(p, _NB)
            if static:
                drain_gathers(p % _NB, rb)
            else:
                for bb in range(_NB):
                    @pl.when(rb == bb)
                    def _():
                        drain_gathers(bb, rb)

            def seg_body(j, carry):
                wrows = [w_v[rb, j, pl.ds(c * P, P)] for c in range(8)]
                rowp = (rb * _SPP + j) * (8 * P) + iot
                acc = [None] * D
                for c in range(8):
                    rowc = rowp + c * P
                    for d in range(D):
                        gv = plsc.load_gather(
                            gath_v, [rowc, jnp.full((16,), d, jnp.int32)])
                        if c == 0:
                            acc[d] = wrows[0] * gv
                        else:
                            acc[d] = acc[d] + wrows[c] * gv
                ob = j * (P * D) + iot16
                for d in range(D):
                    plsc.store_scatter(out_v, [ob + d], acc[d])
                return carry

            lax.fori_loop(0, _SPP, seg_body, 0)

        def out_slice(p):
            return out_hbm.at[pl.ds((seg0_t + p * _SPP) * (P * D),
                                    _SPP * P * D)]

        issue_params(0)
        wait_params(0)
        gen(0)
        issue_params(1)
        wait_params(1)
        gen(1)
        issue_params(2)

        def pipe_body(f, carry):
            @pl.when(f + 2 < F)
            def _():
                wait_params(f + 2)
                gen(f + 2)

            @pl.when(f + 3 < F)
            def _():
                issue_params(f + 3)

            @pl.when(f >= 1)
            def _():
                pltpu.make_async_copy(out_v, out_slice(f - 1), osem).wait()

            combine(f)
            pltpu.async_copy(out_v, out_slice(f), osem)
            return carry

        lax.fori_loop(0, F, pipe_body, 0)
        pltpu.make_async_copy(out_v, out_slice(F - 1), osem).wait()

    return k(inp_t, hist_flat, bbox_emb, params)


def kernel(inp, history, bbox_emb, nodes_min, nodes_extent, bbox_dims):
    n_rays, n_points, _ = inp.shape
    n_nodes = nodes_min.shape[0]
    emb_flat = jax.lax.optimization_barrier(bbox_emb.reshape(-1))
    bbox_emb = emb_flat.reshape(bbox_emb.shape)
    inp_t = jnp.transpose(inp, (0, 2, 1))
    hist_flat = history.reshape(-1).astype(jnp.int32)
    params = jnp.concatenate(
        [nodes_min, nodes_extent, bbox_dims,
         jnp.arange(n_nodes, dtype=jnp.float32)[:, None],
         jnp.zeros((n_nodes, 6), jnp.float32)], axis=1)
    out = _sc_encode(inp_t, hist_flat, bbox_emb, params)
    return out.reshape(n_rays, 4 * n_points * bbox_emb.shape[1])


# speedup vs baseline: 6.6235x; 1.0168x over previous
import functools

import jax
import jax.numpy as jnp
from jax import lax
from jax.experimental import pallas as pl
from jax.experimental.pallas import tpu as pltpu
from jax.experimental.pallas import tpu_sc as plsc

_PI1 = 774363409
_PI2 = -1640531535
_PI3 = 100000007

_NC = 2
_NS = 16
_NW = _NC * _NS

_SPP = 8
_NB = 4


def _gen_kernel(inp_t, hist_flat, params, table_size):
    R, _, P = inp_t.shape
    rpw = R // _NW
    F = rpw * 4 // _SPP
    pi1 = jnp.int32(_PI1)
    pi2 = jnp.int32(_PI2)
    pi3 = jnp.int32(_PI3)
    mask = jnp.int32(table_size - 1)
    mesh = plsc.VectorSubcoreMesh(core_axis_name="c", subcore_axis_name="s",
                                  num_cores=_NC, num_subcores=_NS)

    @functools.partial(
        pl.kernel,
        out_type=(jax.ShapeDtypeStruct((R * 4 * 8 * P,), jnp.int32),
                  jax.ShapeDtypeStruct((R * 4 * 8 * P,), jnp.float32)),
        mesh=mesh,
        scratch_types=[
            pltpu.VMEM((rpw, 3, P), jnp.float32),
            pltpu.VMEM((rpw * 4,), jnp.int32),
            pltpu.VMEM((2, _SPP, 16), jnp.float32),
            pltpu.VMEM((2, _SPP * 8 * P), jnp.int32),
            pltpu.VMEM((2, _SPP * 8 * P), jnp.float32),
            pltpu.SemaphoreType.DMA,
            pltpu.SemaphoreType.DMA,
            pltpu.SemaphoreType.DMA,
        ],
        compiler_params=pltpu.CompilerParams(use_tc_tiling_on_sc=False,
                                             needs_layout_passes=False),
    )
    def k(inp_hbm, hist_hbm, par_hbm, oidx_hbm, ow_hbm,
          inp_all, hist_all, params_v, idx_v, w_v, psem, isem, wsem):
        wid = lax.axis_index("s") * _NC + lax.axis_index("c")
        ray0_t = wid * rpw
        seg0_t = ray0_t * 4

        pltpu.sync_copy(inp_hbm.at[pl.ds(ray0_t, rpw)], inp_all)
        pltpu.sync_copy(hist_hbm.at[pl.ds(seg0_t, rpw * 4)], hist_all)

        def pslot(p):
            return p % 2 if isinstance(p, int) else lax.bitwise_and(p, 1)

        def issue_params(p):
            pltpu.async_copy(par_hbm.at[hist_all.at[pl.ds(p * _SPP, _SPP)]],
                             params_v.at[pslot(p)], psem)

        def wait_params(p):
            pltpu.make_async_copy(
                par_hbm.at[hist_all.at[pl.ds(p * _SPP, _SPP)]],
                params_v.at[pslot(p)], psem).wait()

        def stage_slice(hbm, p):
            return hbm.at[pl.ds((seg0_t + p * _SPP) * (8 * P), _SPP * 8 * P)]

        def gen(p):
            rb = pslot(p)

            def seg_body(j, carry):
                s = p * _SPP + j
                rl = lax.shift_right_logical(s, 2)
                pr = params_v[rb, j, :]
                mn0 = pr[0]
                mn1 = pr[1]
                mn2 = pr[2]
                ex0 = pr[3]
                ex1 = pr[4]
                ex2 = pr[5]
                bd0 = pr[6]
                bd1 = pr[7]
                bd2 = pr[8]
                b = pr[9].astype(jnp.int32)
                px = inp_all[rl, 0, :]
                py = inp_all[rl, 1, :]
                pz = inp_all[rl, 2, :]

                def axis(pp, mn, ex, bd):
                    v = (pp - mn) / ex
                    v = jnp.clip(v, 1e-6, 1.0 - 1e-6) * bd
                    i0 = v.astype(jnp.int32)
                    f0 = i0.astype(jnp.float32)
                    fr = v - f0
                    i1 = jnp.where(v > f0, i0 + 1, i0)
                    return i0, i1, fr

                x0, x1, fx = axis(px, mn0, ex0, bd0)
                y0, y1, fy = axis(py, mn1, ex1, bd1)
                z0, z1, fz = axis(pz, mn2, ex2, bd2)
                hy0 = y0 * pi1
                hy1 = y1 * pi1
                zb0 = (z0 * pi2) ^ (b * pi3)
                zb1 = (z1 * pi2) ^ (b * pi3)
                t00 = x0 ^ hy0
                t01 = x0 ^ hy1
                t10 = x1 ^ hy0
                t11 = x1 ^ hy1
                wx0 = 1.0 - fx
                wy0 = 1.0 - fy
                wz0 = 1.0 - fz
                wxy00 = wx0 * wy0
                wxy01 = wx0 * fy
                wxy10 = fx * wy0
                wxy11 = fx * fy
                idxs = (t00 ^ zb0, t00 ^ zb1, t01 ^ zb0, t01 ^ zb1,
                        t10 ^ zb0, t10 ^ zb1, t11 ^ zb0, t11 ^ zb1)
                ws = (wxy00 * wz0, wxy00 * fz, wxy01 * wz0, wxy01 * fz,
                      wxy10 * wz0, wxy10 * fz, wxy11 * wz0, wxy11 * fz)
                jb = j * (8 * P)
                for c in range(8):
                    idx_v[rb, pl.ds(jb + c * P, P)] = idxs[c] & mask
                    w_v[rb, pl.ds(jb + c * P, P)] = ws[c]
                return carry

            lax.fori_loop(0, _SPP, seg_body, 0)

        issue_params(0)
        wait_params(0)
        gen(0)
        pltpu.async_copy(idx_v.at[0], stage_slice(oidx_hbm, 0), isem)
        pltpu.async_copy(w_v.at[0], stage_slice(ow_hbm, 0), wsem)
        issue_params(1)

        def body(f, carry):
            @pl.when(f + 1 < F)
            def _():
                wait_params(f + 1)
                gen(f + 1)

            @pl.when(f + 2 < F)
            def _():
                issue_params(f + 2)

            pltpu.make_async_copy(idx_v.at[pslot(f)],
                                  stage_slice(oidx_hbm, f), isem).wait()
            pltpu.make_async_copy(w_v.at[pslot(f)],
                                  stage_slice(ow_hbm, f), wsem).wait()

            @pl.when(f + 1 < F)
            def _():
                pltpu.async_copy(idx_v.at[pslot(f + 1)],
                                 stage_slice(oidx_hbm, f + 1), isem)
                pltpu.async_copy(w_v.at[pslot(f + 1)],
                                 stage_slice(ow_hbm, f + 1), wsem)
            return carry

        lax.fori_loop(0, F, body, 0)

    return k(inp_t, hist_flat, params)


def _lookup_kernel(bbox_emb, idx_all, w_all, R, P):
    T, D = bbox_emb.shape
    rpw = R // _NW
    F = rpw * 4 // _SPP
    mesh = plsc.VectorSubcoreMesh(core_axis_name="c", subcore_axis_name="s",
                                  num_cores=_NC, num_subcores=_NS)

    @functools.partial(
        pl.kernel,
        out_type=jax.ShapeDtypeStruct((R * 4 * P * D,), jnp.float32),
        mesh=mesh,
        scratch_types=[
            pltpu.VMEM((_NB, _SPP * 8 * P), jnp.int32),
            pltpu.VMEM((_NB, _SPP * 8 * P), jnp.float32),
            pltpu.VMEM((_NB * _SPP * 8 * P, D), jnp.float32),
            pltpu.VMEM((_SPP * P * D,), jnp.float32),
            pltpu.SemaphoreType.DMA,
            pltpu.SemaphoreType.DMA,
            pltpu.SemaphoreType.DMA,
            pltpu.SemaphoreType.DMA,
            pltpu.SemaphoreType.DMA,
            pltpu.SemaphoreType.DMA,
        ],
        compiler_params=pltpu.CompilerParams(use_tc_tiling_on_sc=False,
                                             needs_layout_passes=False),
    )
    def k(emb_hbm, iall_hbm, wall_hbm, out_hbm,
          idx_v, w_v, gath_v, out_v,
          sem0, sem1, sem2, sem3, insem, osem):
        wid = lax.axis_index("s") * _NC + lax.axis_index("c")
        seg0_t = wid * rpw * 4
        sems = [sem0, sem1, sem2, sem3]
        iot = lax.iota(jnp.int32, 16)
        iot16 = iot * D

        def rslot(p):
            return p % _NB if isinstance(p, int) else lax.rem(p, _NB)

        def stage_slice(hbm, p):
            return hbm.at[pl.ds((seg0_t + p * _SPP) * (8 * P), _SPP * 8 * P)]

        def fire(p):
            rb = rslot(p)

            def seg_body(j, carry):
                src = emb_hbm.at[idx_v.at[rb, pl.ds(j * (8 * P), 8 * P)]]
                dst = gath_v.at[pl.ds((rb * _SPP + j) * (8 * P), 8 * P)]
                if isinstance(p, int):
                    pltpu.async_copy(src, dst, sems[rb])
                else:
                    for bb in range(_NB):
                        @pl.when(rb == bb)
                        def _():
                            pltpu.async_copy(src, dst, sems[bb])
                return carry

            lax.fori_loop(0, _SPP, seg_body, 0)

        def combine(p):
            rb = rslot(p)

            def drain(bb, rbv):
                for j in range(_SPP):
                    pltpu.make_async_copy(
                        emb_hbm.at[idx_v.at[rbv, pl.ds(j * (8 * P), 8 * P)]],
                        gath_v.at[pl.ds((rbv * _SPP + j) * (8 * P), 8 * P)],
                        sems[bb]).wait()

            if isinstance(p, int):
                drain(p % _NB, rb)
            else:
                for bb in range(_NB):
                    @pl.when(rb == bb)
                    def _():
                        drain(bb, rb)

            def seg_body(j, carry):
                jb = j * (8 * P)
                wrows = [w_v[rb, pl.ds(jb + c * P, P)] for c in range(8)]
                rowp = (rb * _SPP + j) * (8 * P) + iot
                acc = [None] * D
                for c in range(8):
                    rowc = rowp + c * P
                    for d in range(D):
                        gv = plsc.load_gather(
                            gath_v, [rowc, jnp.full((16,), d, jnp.int32)])
                        if c == 0:
                            acc[d] = wrows[0] * gv
                        else:
                            acc[d] = acc[d] + wrows[c] * gv
                ob = j * (P * D) + iot16
                for d in range(D):
                    plsc.store_scatter(out_v, [ob + d], acc[d])
                return carry

            lax.fori_loop(0, _SPP, seg_body, 0)

        def out_slice(p):
            return out_hbm.at[pl.ds((seg0_t + p * _SPP) * (P * D),
                                    _SPP * P * D)]

        for p in (0, 1):
            rb = p % _NB
            pltpu.sync_copy(stage_slice(iall_hbm, p),
                            idx_v.at[rb])
            pltpu.sync_copy(stage_slice(wall_hbm, p),
                            w_v.at[rb])
            fire(p)

        def body(f, carry):
            @pl.when(f + 2 < F)
            def _():
                rb2 = lax.rem(f + 2, _NB)
                pltpu.make_async_copy(
                    stage_slice(iall_hbm, f + 2),
                    idx_v.at[rb2], insem).wait()
                pltpu.make_async_copy(
                    stage_slice(wall_hbm, f + 2),
                    w_v.at[rb2], insem).wait()
                fire(f + 2)

            @pl.when(f + 3 < F)
            def _():
                rb3 = lax.rem(f + 3, _NB)
                pltpu.async_copy(stage_slice(iall_hbm, f + 3),
                                 idx_v.at[rb3], insem)
                pltpu.async_copy(stage_slice(wall_hbm, f + 3),
                                 w_v.at[rb3], insem)

            @pl.when(f >= 1)
            def _():
                pltpu.make_async_copy(out_v, out_slice(f - 1), osem).wait()

            combine(f)
            pltpu.async_copy(out_v, out_slice(f), osem)
            return carry

        @pl.when(2 < F)
        def _():
            rb2 = 2 % _NB
            pltpu.async_copy(stage_slice(iall_hbm, 2),
                             idx_v.at[rb2], insem)
            pltpu.async_copy(stage_slice(wall_hbm, 2),
                             w_v.at[rb2], insem)

        lax.fori_loop(0, F, body, 0)
        pltpu.make_async_copy(out_v, out_slice(F - 1), osem).wait()

    return k(bbox_emb, idx_all, w_all)


@functools.partial(jax.jit, static_argnums=(4, 5))
def _sc_encode(inp_t, hist_flat, bbox_emb, params, R, P):
    idx_all, w_all = _gen_kernel(inp_t, hist_flat, params, bbox_emb.shape[0])
    return _lookup_kernel(bbox_emb, idx_all, w_all, R, P)


def kernel(inp, history, bbox_emb, nodes_min, nodes_extent, bbox_dims):
    n_rays, n_points, _ = inp.shape
    n_nodes = nodes_min.shape[0]
    emb_flat = jax.lax.optimization_barrier(bbox_emb.reshape(-1))
    bbox_emb = emb_flat.reshape(bbox_emb.shape)
    inp_t = jnp.transpose(inp, (0, 2, 1))
    hist_flat = history.reshape(-1).astype(jnp.int32)
    params = jnp.concatenate(
        [nodes_min, nodes_extent, bbox_dims,
         jnp.arange(n_nodes, dtype=jnp.float32)[:, None],
         jnp.zeros((n_nodes, 6), jnp.float32)], axis=1)
    out = _sc_encode(inp_t, hist_flat, bbox_emb, params, n_rays, n_points)
    return out.reshape(n_rays, 4 * n_points * bbox_emb.shape[1])
